# Initial kernel scaffold; baseline (speedup 1.0000x reference)
import jax
import jax.numpy as jnp
from jax.experimental import pallas as pl


def kernel(x, edge_index, W0, b0, W1, b1):
    raise NotImplementedError("write your pallas kernel here")


# speedup vs baseline: 5.4560x; 5.4560x over previous
import jax
import jax.numpy as jnp
from jax import lax
from jax.experimental import pallas as pl
from jax.experimental.pallas import tpu as pltpu
from jax.experimental.pallas import tpu_sc as plsc

_EMB = 128
_R = 10240
_NC, _NS = 2, 16
_NW = _NC * _NS
_CH = 128
_NCH = 80
_EPAD = _NW * _NCH * _CH
_DUMMY = 10000
_NPH = 2
_PH_R = _R // _NPH
_ACC_R = 5376
_ACC_TILE = _ACC_R // _NS
_SPARE = _PH_R


def _sc_propagate(table, gidx, sidx_ph, sidx_orig=None):
    with_degrees = sidx_orig is not None
    mesh = plsc.VectorSubcoreMesh(core_axis_name="c", subcore_axis_name="s")
    out_type = [jax.ShapeDtypeStruct((_NPH, _NC, _ACC_R, _EMB), jnp.float32)]
    if with_degrees:
        out_type.append(jax.ShapeDtypeStruct((_NW, _R), jnp.float32))
        out_type.append(jax.ShapeDtypeStruct((_NW, _R), jnp.float32))
    scratch = [
        pltpu.VMEM((_NCH, _CH), jnp.int32),
        pltpu.VMEM((_NCH, _CH), jnp.int32),
        pltpu.VMEM((_CH, _EMB), jnp.float32),
        pltpu.VMEM((_CH, _EMB), jnp.float32),
        pltpu.VMEM((16, _EMB), jnp.float32),
        pltpu.VMEM_SHARED((_ACC_R, _EMB), jnp.float32),
        pltpu.SemaphoreType.DMA,
        pltpu.SemaphoreType.DMA,
    ]
    if with_degrees:
        scratch.append(pltpu.VMEM((_NCH, _CH), jnp.int32))
        scratch.append(pltpu.VMEM((_R,), jnp.float32))
        scratch.append(pltpu.VMEM((_R,), jnp.float32))

    def body(*refs):
        if with_degrees:
            (table_hbm, gidx_hbm, sidx_ph_hbm, sidx_o_hbm,
             acc_out, dn_out, be_out,
             gi, si, bufa, bufb, zblk, acc, sema, semb, so, dnv, bev) = refs
        else:
            (table_hbm, gidx_hbm, sidx_ph_hbm,
             acc_out, gi, si, bufa, bufb, zblk, acc, sema, semb) = refs
        cid = lax.axis_index("c")
        sid = lax.axis_index("s")
        w = cid * _NS + sid

        pltpu.sync_copy(gidx_hbm.at[w], gi)

        z16 = jnp.zeros((16,), jnp.float32)
        for r in range(16):
            for q in range(_EMB // 16):
                zblk[r, pl.ds(q * 16, 16)] = z16

        if with_degrees:
            pltpu.sync_copy(sidx_o_hbm.at[w], so)

            def zero_deg(i, carry):
                dnv[pl.ds(i * 16, 16)] = z16
                bev[pl.ds(i * 16, 16)] = z16
                return carry
            lax.fori_loop(0, _R // 16, zero_deg, 0)

        base = sid * _ACC_TILE
        ones16 = jnp.ones((16,), jnp.float32)

        for p in range(_NPH):
            pltpu.sync_copy(sidx_ph_hbm.at[p, w], si)

            def zero_acc(i, carry):
                pltpu.sync_copy(zblk, acc.at[pl.ds(base + i * 16, 16)])
                return carry
            lax.fori_loop(0, _ACC_TILE // 16, zero_acc, 0)

            plsc.subcore_barrier()

            do_deg = with_degrees and p == 0

            def step(i, carry):
                j = i * 2
                ga = pltpu.async_copy(table_hbm.at[gi.at[j]], bufa, sema)
                gb = pltpu.async_copy(table_hbm.at[gi.at[j + 1]], bufb, semb)
                ga.wait()
                sa = pltpu.async_copy(bufa, acc.at[si.at[j]], sema, add=True)
                gb.wait()
                sb = pltpu.async_copy(bufb, acc.at[si.at[j + 1]], semb,
                                      add=True)
                if do_deg:
                    for jo in range(2):
                        for v in range(_CH // 16):
                            plsc.addupdate_scatter(
                                dnv, [gi[j + jo, pl.ds(v * 16, 16)]], ones16)
                            plsc.addupdate_scatter(
                                bev, [so[j + jo, pl.ds(v * 16, 16)]], ones16)
                sa.wait()
                sb.wait()
                return carry
            lax.fori_loop(0, _NCH // 2, step, 0)

            plsc.subcore_barrier()
            pltpu.sync_copy(acc.at[pl.ds(base, _ACC_TILE)],
                            acc_out.at[p, cid, pl.ds(base, _ACC_TILE)])

        if with_degrees:
            pltpu.sync_copy(dnv, dn_out.at[w])
            pltpu.sync_copy(bev, be_out.at[w])

    fn = pl.kernel(body, mesh=mesh, out_type=out_type, scratch_types=scratch,
                   compiler_params=pltpu.CompilerParams(
                       needs_layout_passes=False))
    if with_degrees:
        return fn(table, gidx, sidx_ph, sidx_orig)
    outs = fn(table, gidx, sidx_ph)
    return outs[0] if isinstance(outs, (list, tuple)) else outs


_BR = 1024


def _tc_combine_scale(parts, deg_parts):
    def body(p_ref, d_ref, o_ref):
        deg = jnp.sum(d_ref[...], axis=0)
        inv = jnp.where(deg > 0, 1.0 / deg, 0.0)
        o_ref[...] = (p_ref[0, 0] + p_ref[0, 1]) * inv[:, None]

    nb = _PH_R // _BR
    return pl.pallas_call(
        body,
        grid=(_NPH, nb),
        in_specs=[
            pl.BlockSpec((1, _NC, _BR, _EMB), lambda p, i: (p, 0, i, 0)),
            pl.BlockSpec((_NW, _BR), lambda p, i: (0, p * nb + i)),
        ],
        out_specs=pl.BlockSpec((_BR, _EMB), lambda p, i: (p * nb + i, 0)),
        out_shape=jax.ShapeDtypeStruct((_R, _EMB), jnp.float32),
    )(parts, deg_parts)


def _tc_finalize(z_parts, dn_parts, w0, b0, w1, b1):
    def body(zp, dp, w0r, b0r, w1r, b1r, o_ref):
        deg = jnp.sum(dp[...], axis=0)
        inv = jnp.where(deg > 0, 1.0 / deg, 0.0)
        z = (zp[0, 0] + zp[0, 1]) * inv[:, None]
        h0 = jnp.dot(z, w0r[...], preferred_element_type=jnp.float32) + b0r[...]
        h0 = h0 / jnp.maximum(
            jnp.sqrt(jnp.sum(h0 * h0, axis=-1, keepdims=True)), 1e-12)
        h1 = jnp.dot(z, w1r[...], preferred_element_type=jnp.float32) + b1r[...]
        h1 = h1 / jnp.maximum(
            jnp.sqrt(jnp.sum(h1 * h1, axis=-1, keepdims=True)), 1e-12)
        o_ref[...] = h0 + h1

    nb = _PH_R // _BR
    return pl.pallas_call(
        body,
        grid=(_NPH, nb),
        in_specs=[
            pl.BlockSpec((1, _NC, _BR, _EMB), lambda p, i: (p, 0, i, 0)),
            pl.BlockSpec((_NW, _BR), lambda p, i: (0, p * nb + i)),
            pl.BlockSpec((_EMB, _EMB), lambda p, i: (0, 0)),
            pl.BlockSpec((1, _EMB), lambda p, i: (0, 0)),
            pl.BlockSpec((_EMB, _EMB), lambda p, i: (0, 0)),
            pl.BlockSpec((1, _EMB), lambda p, i: (0, 0)),
        ],
        out_specs=pl.BlockSpec((_BR, _EMB), lambda p, i: (p * nb + i, 0)),
        out_shape=jax.ShapeDtypeStruct((_R, _EMB), jnp.float32),
    )(z_parts, dn_parts, w0, b0, w1, b1)


def _phase_split(sidx):
    locs = []
    for p in range(_NPH):
        lo = p * _PH_R
        inr = (sidx >= lo) & (sidx < lo + _PH_R)
        locs.append(jnp.where(inr, sidx - lo, _SPARE).astype(jnp.int32))
    return jnp.stack(locs).reshape(_NPH, _NW, _NCH, _CH)


def kernel(x, edge_index, W0, b0, W1, b1):
    node_idx = edge_index[0]
    hedge_idx = edge_index[1]
    n = x.shape[0]
    e = node_idx.shape[0]
    padlen = _EPAD - e
    pad = jnp.full((padlen,), _DUMMY, jnp.int32)
    nidx = jnp.concatenate([node_idx, pad])
    hidx = jnp.concatenate([hedge_idx, pad])
    nidx_r = nidx.reshape(_NW, _NCH, _CH)
    hidx_r = hidx.reshape(_NW, _NCH, _CH)
    xp = jnp.zeros((_R, _EMB), jnp.float32).at[:n].set(x)

    he_parts, dn_parts, be_parts = _sc_propagate(
        xp, nidx_r, _phase_split(hidx), sidx_orig=hidx_r)
    he = _tc_combine_scale(he_parts, be_parts)
    z_parts = _sc_propagate(he, hidx_r, _phase_split(nidx))
    out = _tc_finalize(z_parts, dn_parts,
                       W0, b0.reshape(1, _EMB), W1, b1.reshape(1, _EMB))
    return out[:n]


# speedup vs baseline: 5.5818x; 1.0231x over previous
import jax
import jax.numpy as jnp
from jax import lax
from jax.experimental import pallas as pl
from jax.experimental.pallas import tpu as pltpu
from jax.experimental.pallas import tpu_sc as plsc

_EMB = 128
_R = 10240
_NC, _NS = 2, 16
_NW = _NC * _NS
_CH = 128
_NCH = 80
_EPAD = _NW * _NCH * _CH
_DUMMY = 10000
_G = 8
_NG = _NCH // _G

_P0 = 8640
_ACC0 = 8704
_ACC0_TILE = _ACC0 // _NS
_P1R = _R - _P0
_ACC1 = 1664
_ACC1_TILE = _ACC1 // _NS
_CAP1 = (_NCH + 4) * _CH
_OUT_R = _ACC0 + _ACC1
_SHIFT = 16384


def _sc_propagate(table, gidx, sidx):
    mesh = plsc.VectorSubcoreMesh(core_axis_name="c", subcore_axis_name="s")
    out_type = [
        jax.ShapeDtypeStruct((_NC, _OUT_R, _EMB), jnp.float32),
        jax.ShapeDtypeStruct((_NW, _R), jnp.float32),
    ]
    scratch = [
        pltpu.VMEM((2, _G, _CH), jnp.int32),
        pltpu.VMEM((2, _G, _CH), jnp.int32),
        pltpu.VMEM((_CAP1,), jnp.int32),
        pltpu.VMEM((8, _CH), jnp.int32),
        pltpu.VMEM((8, _CH), jnp.int32),
        pltpu.VMEM((_CH, _EMB), jnp.float32),
        pltpu.VMEM((_CH, _EMB), jnp.float32),
        pltpu.VMEM_SHARED((_ACC0, _EMB), jnp.float32),
        pltpu.SemaphoreType.DMA,
        pltpu.SemaphoreType.DMA,
        pltpu.SemaphoreType.DMA,
        pltpu.VMEM((_R,), jnp.float32),
    ]

    def body(table_hbm, gidx_hbm, sidx_hbm, acc_out, hist_out,
             gib, sgb, cw, grow, csrow, bufa, bufb, acc,
             sema, semb, semc, histv):
        cid = lax.axis_index("c")
        sid = lax.axis_index("s")
        w = cid * _NS + sid

        z16 = jnp.zeros((16,), jnp.float32)

        def zero_hist(i, carry):
            histv[pl.ds(i * 16, 16)] = z16
            return carry
        lax.fori_loop(0, _R // 16, zero_hist, 0)

        for r in range(16):
            for q in range(_EMB // 16):
                bufa[r, pl.ds(q * 16, 16)] = z16

        base0 = sid * _ACC0_TILE

        def zero_acc0(i, carry):
            pltpu.sync_copy(bufa.at[pl.ds(0, 8)],
                            acc.at[pl.ds(base0 + i * 8, 8)])
            return carry
        lax.fori_loop(0, _ACC0_TILE // 8, zero_acc0, 0)

        pltpu.sync_copy(gidx_hbm.at[w, pl.ds(0, _G)], gib.at[0])
        pltpu.sync_copy(sidx_hbm.at[w, pl.ds(0, _G)], sgb.at[0])

        plsc.subcore_barrier()

        ones16 = jnp.ones((16,), jnp.float32)
        p0 = jnp.full((16,), _P0, jnp.int32)

        def group(g, off):
            cur = g % 2
            nxt = (g + 1) % 2
            gnext = jnp.minimum(g + 1, _NG - 1)
            pg = pltpu.async_copy(gidx_hbm.at[w, pl.ds(gnext * _G, _G)],
                                  gib.at[nxt], semc)
            ps = pltpu.async_copy(sidx_hbm.at[w, pl.ds(gnext * _G, _G)],
                                  sgb.at[nxt], semc)
            for pp in range(_G // 2):
                j = pp * 2
                ga = pltpu.async_copy(table_hbm.at[gib.at[cur, j]],
                                      bufa, sema)
                gb = pltpu.async_copy(table_hbm.at[gib.at[cur, j + 1]],
                                      bufb, semb)
                for jo in range(2):
                    for v in range(_CH // 16):
                        sv = sgb[cur, j + jo, pl.ds(v * 16, 16)]
                        gv = gib[cur, j + jo, pl.ds(v * 16, 16)]
                        plsc.addupdate_scatter(histv, [sv], ones16)
                        sgb[cur, j + jo, pl.ds(v * 16, 16)] = (
                            jnp.minimum(sv, p0))
                        m = sv >= p0
                        pk = gv + (sv - p0) * _SHIFT
                        plsc.store_compressed(cw.at[pl.ds(off, 16)], pk,
                                              mask=m)
                        off = off + jnp.sum(m.astype(jnp.int32))
                ga.wait()
                sa = pltpu.async_copy(bufa, acc.at[sgb.at[cur, j]],
                                      sema, add=True)
                gb.wait()
                sb = pltpu.async_copy(bufb, acc.at[sgb.at[cur, j + 1]],
                                      semb, add=True)
                sa.wait()
                sb.wait()
            pg.wait()
            ps.wait()
            return off
        off = lax.fori_loop(0, _NG, group, jnp.int32(0))

        pltpu.sync_copy(histv, hist_out.at[w])

        plsc.subcore_barrier()
        pltpu.sync_copy(acc.at[pl.ds(base0, _ACC0_TILE)],
                        acc_out.at[cid, pl.ds(base0, _ACC0_TILE)])
        plsc.subcore_barrier()

        pad16 = jnp.full((16,), _P1R * _SHIFT, jnp.int32)
        for k in range(9):
            cw[pl.ds(off + k * 16, 16)] = pad16
        nch1 = (off + _CH - 1) // _CH

        for r in range(16):
            for q in range(_EMB // 16):
                bufa[r, pl.ds(q * 16, 16)] = z16

        base1 = sid * _ACC1_TILE

        def zero_acc1(i, carry):
            pltpu.sync_copy(bufa.at[pl.ds(0, 8)],
                            acc.at[pl.ds(base1 + i * 8, 8)])
            return carry
        lax.fori_loop(0, _ACC1_TILE // 8, zero_acc1, 0)

        plsc.subcore_barrier()

        mask14 = jnp.full((16,), _SHIFT - 1, jnp.int32)

        def step1(j, carry):
            for v in range(_CH // 16):
                pk = cw[pl.ds(j * _CH + v * 16, 16)]
                grow[0, pl.ds(v * 16, 16)] = pk & mask14
                csrow[0, pl.ds(v * 16, 16)] = pk // _SHIFT
            ga = pltpu.async_copy(table_hbm.at[grow.at[0]], bufa, sema)
            ga.wait()
            sa = pltpu.async_copy(bufa, acc.at[csrow.at[0]], sema, add=True)
            sa.wait()
            return carry
        lax.fori_loop(0, nch1, step1, 0)

        plsc.subcore_barrier()
        pltpu.sync_copy(acc.at[pl.ds(base1, _ACC1_TILE)],
                        acc_out.at[cid, pl.ds(_ACC0 + base1, _ACC1_TILE)])

    fn = pl.kernel(body, mesh=mesh, out_type=out_type, scratch_types=scratch,
                   compiler_params=pltpu.CompilerParams(
                       needs_layout_passes=False))
    return fn(table, gidx, sidx)


def _tc_combine_scale(parts, deg_parts):
    def body(p_ref, d_ref, o_ref):
        deg = jnp.sum(d_ref[...], axis=0)
        inv = jnp.where(deg > 0, 1.0 / deg, 0.0)
        comb0 = p_ref[0, :_P0] + p_ref[1, :_P0]
        comb1 = (p_ref[0, _ACC0:_ACC0 + _P1R] + p_ref[1, _ACC0:_ACC0 + _P1R])
        o_ref[:_P0] = comb0 * inv[:_P0, None]
        o_ref[_P0:] = comb1 * inv[_P0:, None]

    return pl.pallas_call(
        body,
        out_shape=jax.ShapeDtypeStruct((_R, _EMB), jnp.float32),
    )(parts, deg_parts)


def _tc_finalize(z_parts, dn_parts, w0, b0, w1, b1):
    def body(zp, dp, w0r, b0r, w1r, b1r, o_ref):
        deg = jnp.sum(dp[...], axis=0)
        inv = jnp.where(deg > 0, 1.0 / deg, 0.0)
        z0 = (zp[0, :_P0] + zp[1, :_P0]) * inv[:_P0, None]
        z1 = ((zp[0, _ACC0:_ACC0 + _P1R] + zp[1, _ACC0:_ACC0 + _P1R])
              * inv[_P0:, None])
        z = jnp.concatenate([z0, z1], axis=0)
        h0 = jnp.dot(z, w0r[...], preferred_element_type=jnp.float32) + b0r[...]
        h0 = h0 / jnp.maximum(
            jnp.sqrt(jnp.sum(h0 * h0, axis=-1, keepdims=True)), 1e-12)
        h1 = jnp.dot(z, w1r[...], preferred_element_type=jnp.float32) + b1r[...]
        h1 = h1 / jnp.maximum(
            jnp.sqrt(jnp.sum(h1 * h1, axis=-1, keepdims=True)), 1e-12)
        o_ref[...] = h0 + h1

    return pl.pallas_call(
        body,
        out_shape=jax.ShapeDtypeStruct((_R, _EMB), jnp.float32),
    )(z_parts, dn_parts, w0, b0, w1, b1)


def kernel(x, edge_index, W0, b0, W1, b1):
    node_idx = edge_index[0]
    hedge_idx = edge_index[1]
    n = x.shape[0]
    e = node_idx.shape[0]
    padlen = _EPAD - e
    pad = jnp.full((padlen,), _DUMMY, jnp.int32)
    nidx = jnp.concatenate([node_idx, pad]).reshape(_NW, _NCH, _CH)
    hidx = jnp.concatenate([hedge_idx, pad]).reshape(_NW, _NCH, _CH)
    xp = jnp.zeros((_R, _EMB), jnp.float32).at[:n].set(x)

    he_parts, be_parts = _sc_propagate(xp, nidx, hidx)
    he = _tc_combine_scale(he_parts, be_parts)
    z_parts, dn_parts = _sc_propagate(he, hidx, nidx)
    out = _tc_finalize(z_parts, dn_parts,
                       W0, b0.reshape(1, _EMB), W1, b1.reshape(1, _EMB))
    return out[:n]


# speedup vs baseline: 5.6068x; 1.0045x over previous
import jax
import jax.numpy as jnp
from jax import lax
from jax.experimental import pallas as pl
from jax.experimental.pallas import tpu as pltpu
from jax.experimental.pallas import tpu_sc as plsc

_EMB = 128
_R = 10240
_NC, _NS = 2, 16
_NW = _NC * _NS
_CH = 128
_NCH = 80
_EPAD = _NW * _NCH * _CH
_DUMMY = 10000
_G = 8
_NG = _NCH // _G

_P0 = 8640
_ACC0 = 8704
_ACC0_TILE = _ACC0 // _NS
_P1R = _R - _P0
_ACC1 = 1664
_ACC1_TILE = _ACC1 // _NS
_CAP1 = (_NCH + 4) * _CH
_OUT_R = _ACC0 + _ACC1
_SHIFT = 16384


def _sc_propagate(table, gidx, sidx):
    mesh = plsc.VectorSubcoreMesh(core_axis_name="c", subcore_axis_name="s")
    out_type = [
        jax.ShapeDtypeStruct((_NC, _OUT_R, _EMB), jnp.float32),
        jax.ShapeDtypeStruct((_NW, _R), jnp.float32),
    ]
    scratch = [
        pltpu.VMEM((2, _G, _CH), jnp.int32),
        pltpu.VMEM((2, _G, _CH), jnp.int32),
        pltpu.VMEM((_CAP1,), jnp.int32),
        pltpu.VMEM((8, _CH), jnp.int32),
        pltpu.VMEM((8, _CH), jnp.int32),
        pltpu.VMEM((_CH, _EMB), jnp.float32),
        pltpu.VMEM((_CH, _EMB), jnp.float32),
        pltpu.VMEM_SHARED((_ACC0, _EMB), jnp.float32),
        pltpu.SemaphoreType.DMA,
        pltpu.SemaphoreType.DMA,
        pltpu.SemaphoreType.DMA,
        pltpu.VMEM((_R,), jnp.float32),
    ]

    def body(table_hbm, gidx_hbm, sidx_hbm, acc_out, hist_out,
             gib, sgb, cw, grow, csrow, bufa, bufb, acc,
             sema, semb, semc, histv):
        cid = lax.axis_index("c")
        sid = lax.axis_index("s")
        w = cid * _NS + sid

        z16 = jnp.zeros((16,), jnp.float32)

        def zero_hist(i, carry):
            histv[pl.ds(i * 16, 16)] = z16
            return carry
        lax.fori_loop(0, _R // 16, zero_hist, 0)

        def zero_bufb(r, carry):
            for q in range(_EMB // 16):
                bufb[r, pl.ds(q * 16, 16)] = z16
            return carry
        lax.fori_loop(0, _CH, zero_bufb, 0)

        base0 = sid * _ACC0_TILE
        zcps = []
        for t in range(_ACC0_TILE // _CH):
            zcps.append(pltpu.async_copy(
                bufb, acc.at[pl.ds(base0 + t * _CH, _CH)], sema))
        _TAIL0 = _ACC0_TILE % _CH
        if _TAIL0:
            zcps.append(pltpu.async_copy(
                bufb.at[pl.ds(0, _TAIL0)],
                acc.at[pl.ds(base0 + _ACC0_TILE - _TAIL0, _TAIL0)], sema))
        for c in zcps:
            c.wait()

        pltpu.sync_copy(gidx_hbm.at[w, pl.ds(0, _G)], gib.at[0])
        pltpu.sync_copy(sidx_hbm.at[w, pl.ds(0, _G)], sgb.at[0])

        plsc.subcore_barrier()

        ones16 = jnp.ones((16,), jnp.float32)
        p0 = jnp.full((16,), _P0, jnp.int32)

        def group(g, off):
            cur = g % 2
            nxt = (g + 1) % 2
            gnext = jnp.minimum(g + 1, _NG - 1)
            pg = pltpu.async_copy(gidx_hbm.at[w, pl.ds(gnext * _G, _G)],
                                  gib.at[nxt], semc)
            ps = pltpu.async_copy(sidx_hbm.at[w, pl.ds(gnext * _G, _G)],
                                  sgb.at[nxt], semc)
            for pp in range(_G // 2):
                j = pp * 2
                ga = pltpu.async_copy(table_hbm.at[gib.at[cur, j]],
                                      bufa, sema)
                gb = pltpu.async_copy(table_hbm.at[gib.at[cur, j + 1]],
                                      bufb, semb)
                for jo in range(2):
                    for v in range(_CH // 16):
                        sv = sgb[cur, j + jo, pl.ds(v * 16, 16)]
                        gv = gib[cur, j + jo, pl.ds(v * 16, 16)]
                        plsc.addupdate_scatter(histv, [sv], ones16)
                        sgb[cur, j + jo, pl.ds(v * 16, 16)] = (
                            jnp.minimum(sv, p0))
                        m = sv >= p0
                        pk = gv + (sv - p0) * _SHIFT
                        plsc.store_compressed(cw.at[pl.ds(off, 16)], pk,
                                              mask=m)
                        off = off + jnp.sum(m.astype(jnp.int32))
                ga.wait()
                sa = pltpu.async_copy(bufa, acc.at[sgb.at[cur, j]],
                                      sema, add=True)
                gb.wait()
                sb = pltpu.async_copy(bufb, acc.at[sgb.at[cur, j + 1]],
                                      semb, add=True)
                sa.wait()
                sb.wait()
            pg.wait()
            ps.wait()
            return off
        off = lax.fori_loop(0, _NG, group, jnp.int32(0))

        pltpu.sync_copy(histv, hist_out.at[w])

        plsc.subcore_barrier()
        pltpu.sync_copy(acc.at[pl.ds(base0, _ACC0_TILE)],
                        acc_out.at[cid, pl.ds(base0, _ACC0_TILE)])
        plsc.subcore_barrier()

        pad16 = jnp.full((16,), _P1R * _SHIFT, jnp.int32)
        for k in range(9):
            cw[pl.ds(off + k * 16, 16)] = pad16
        nch1 = (off + _CH - 1) // _CH

        def zero_bufb2(r, carry):
            for q in range(_EMB // 16):
                bufb[r, pl.ds(q * 16, 16)] = z16
            return carry
        lax.fori_loop(0, _ACC1_TILE, zero_bufb2, 0)

        base1 = sid * _ACC1_TILE
        pltpu.async_copy(bufb.at[pl.ds(0, _ACC1_TILE)],
                         acc.at[pl.ds(base1, _ACC1_TILE)], sema).wait()

        plsc.subcore_barrier()

        mask14 = jnp.full((16,), _SHIFT - 1, jnp.int32)

        def step1(j, carry):
            for v in range(_CH // 16):
                pk = cw[pl.ds(j * _CH + v * 16, 16)]
                grow[0, pl.ds(v * 16, 16)] = pk & mask14
                csrow[0, pl.ds(v * 16, 16)] = pk // _SHIFT
            ga = pltpu.async_copy(table_hbm.at[grow.at[0]], bufa, sema)
            ga.wait()
            sa = pltpu.async_copy(bufa, acc.at[csrow.at[0]], sema, add=True)
            sa.wait()
            return carry
        lax.fori_loop(0, nch1, step1, 0)

        plsc.subcore_barrier()
        pltpu.sync_copy(acc.at[pl.ds(base1, _ACC1_TILE)],
                        acc_out.at[cid, pl.ds(_ACC0 + base1, _ACC1_TILE)])

    fn = pl.kernel(body, mesh=mesh, out_type=out_type, scratch_types=scratch,
                   compiler_params=pltpu.CompilerParams(
                       needs_layout_passes=False))
    return fn(table, gidx, sidx)


def _tc_combine_scale(parts, deg_parts):
    def body(p_ref, d_ref, o_ref):
        deg = jnp.sum(d_ref[...], axis=0)
        inv = jnp.where(deg > 0, 1.0 / deg, 0.0)
        comb0 = p_ref[0, :_P0] + p_ref[1, :_P0]
        comb1 = (p_ref[0, _ACC0:_ACC0 + _P1R] + p_ref[1, _ACC0:_ACC0 + _P1R])
        o_ref[:_P0] = comb0 * inv[:_P0, None]
        o_ref[_P0:] = comb1 * inv[_P0:, None]

    return pl.pallas_call(
        body,
        out_shape=jax.ShapeDtypeStruct((_R, _EMB), jnp.float32),
    )(parts, deg_parts)


def _tc_finalize(z_parts, dn_parts, w0, b0, w1, b1):
    def body(zp, dp, w0r, b0r, w1r, b1r, o_ref):
        deg = jnp.sum(dp[...], axis=0)
        inv = jnp.where(deg > 0, 1.0 / deg, 0.0)
        z0 = (zp[0, :_P0] + zp[1, :_P0]) * inv[:_P0, None]
        z1 = ((zp[0, _ACC0:_ACC0 + _P1R] + zp[1, _ACC0:_ACC0 + _P1R])
              * inv[_P0:, None])
        z = jnp.concatenate([z0, z1], axis=0)
        h0 = jnp.dot(z, w0r[...], preferred_element_type=jnp.float32) + b0r[...]
        h0 = h0 / jnp.maximum(
            jnp.sqrt(jnp.sum(h0 * h0, axis=-1, keepdims=True)), 1e-12)
        h1 = jnp.dot(z, w1r[...], preferred_element_type=jnp.float32) + b1r[...]
        h1 = h1 / jnp.maximum(
            jnp.sqrt(jnp.sum(h1 * h1, axis=-1, keepdims=True)), 1e-12)
        o_ref[...] = h0 + h1

    return pl.pallas_call(
        body,
        out_shape=jax.ShapeDtypeStruct((_R, _EMB), jnp.float32),
    )(z_parts, dn_parts, w0, b0, w1, b1)


def kernel(x, edge_index, W0, b0, W1, b1):
    node_idx = edge_index[0]
    hedge_idx = edge_index[1]
    n = x.shape[0]
    e = node_idx.shape[0]
    padlen = _EPAD - e
    pad = jnp.full((padlen,), _DUMMY, jnp.int32)
    nidx = jnp.concatenate([node_idx, pad]).reshape(_NW, _NCH, _CH)
    hidx = jnp.concatenate([hedge_idx, pad]).reshape(_NW, _NCH, _CH)
    xp = jnp.zeros((_R, _EMB), jnp.float32).at[:n].set(x)

    he_parts, be_parts = _sc_propagate(xp, nidx, hidx)
    he = _tc_combine_scale(he_parts, be_parts)
    z_parts, dn_parts = _sc_propagate(he, hidx, nidx)
    out = _tc_finalize(z_parts, dn_parts,
                       W0, b0.reshape(1, _EMB), W1, b1.reshape(1, _EMB))
    return out[:n]
